# baseline (speedup 1.0000x reference)
import jax
import jax.numpy as jnp
from jax import lax
from jax.experimental import pallas as pl
from jax.experimental.pallas import tpu as pltpu
from jax.experimental.pallas import tpu_sc as plsc

N_NODES = 100000
MEM_DIM = 64
BATCH = 16384
HALF_LIFE = 40.0
LN2 = 0.69314718

NC, NS, L = 2, 16, 16
NW = NC * NS
ROWS_W = 3128
CHUNK = 256
CP_ROWS = 128
N_GROUPS = BATCH // L


def _sc_body(ids_hbm, ns_hbm, ts_hbm, mem_hbm, lut_hbm,
             memout_hbm, tsout_hbm,
             ids_v, ts_v, lut_v, claim_v, list_v,
             bidx_v, nidx_v, alpha_v, tsn_v,
             old_v, nsr_v, new_v, cp_v, sem0, sem1):
    wid = lax.axis_index("c") * NS + lax.axis_index("s")
    lo = wid * ROWS_W
    nrows = jnp.minimum(ROWS_W, N_NODES - lo)
    lane = lax.iota(jnp.int32, L)

    pltpu.sync_copy(ids_hbm, ids_v)
    pltpu.sync_copy(ts_hbm, ts_v)

    nfull_t = nrows // CP_ROWS
    nrem_t = (nrows % CP_ROWS) // 8

    def lut_in(k, _):
        off = k * CP_ROWS
        pltpu.sync_copy(lut_hbm.at[pl.ds(pl.multiple_of(lo + off, 8), CP_ROWS)],
                        lut_v.at[pl.ds(off, CP_ROWS)])
        return 0
    lax.fori_loop(0, nfull_t, lut_in, 0)

    def lut_in8(j, _):
        off = nfull_t * CP_ROWS + j * 8
        pltpu.sync_copy(lut_hbm.at[pl.ds(pl.multiple_of(lo + off, 8), 8)],
                        lut_v.at[pl.ds(off, 8)])
        return 0
    lax.fori_loop(0, nrem_t, lut_in8, 0)

    def lut_out(k, _):
        off = k * CP_ROWS
        pltpu.sync_copy(lut_v.at[pl.ds(off, CP_ROWS)],
                        tsout_hbm.at[pl.ds(pl.multiple_of(lo + off, 8), CP_ROWS)])
        return 0
    lax.fori_loop(0, nfull_t, lut_out, 0)

    def lut_out8(j, _):
        off = nfull_t * CP_ROWS + j * 8
        pltpu.sync_copy(lut_v.at[pl.ds(off, 8)],
                        tsout_hbm.at[pl.ds(pl.multiple_of(lo + off, 8), 8)])
        return 0
    lax.fori_loop(0, nrem_t, lut_out8, 0)

    def row_cp(k, _):
        off = lo + k * CP_ROWS
        pltpu.sync_copy(mem_hbm.at[pl.ds(off, CP_ROWS), :], cp_v)
        pltpu.sync_copy(cp_v, memout_hbm.at[pl.ds(off, CP_ROWS), :])
        return 0
    lax.fori_loop(0, nfull_t, row_cp, 0)

    def row_cp8(j, _):
        off = lo + nfull_t * CP_ROWS + j * 8
        pltpu.sync_copy(mem_hbm.at[pl.ds(off, 8), :], cp_v.at[pl.ds(0, 8), :])
        pltpu.sync_copy(cp_v.at[pl.ds(0, 8), :], memout_hbm.at[pl.ds(off, 8), :])
        return 0
    lax.fori_loop(0, nrem_t, row_cp8, 0)

    def memset_grp(g, _):
        claim_v[pl.ds(g * L, L)] = jnp.full((L,), -1, jnp.int32)
        return 0
    lax.fori_loop(0, (ROWS_W + L - 1) // L, memset_grp, 0)

    hi = lo + nrows

    def claim_grp(g, _):
        ids16 = ids_v[pl.ds(g * L, L)]
        own = (ids16 >= lo) & (ids16 < hi)
        bvec = g * L + lane
        idsl = jnp.where(own, ids16 - lo, 0)

        w = plsc.load_gather(claim_v, [idsl])
        m = own & (bvec > w)
        plsc.store_scatter(claim_v, [idsl], bvec, mask=m)
        w2 = plsc.load_gather(claim_v, [idsl])
        pending = jnp.sum((own & (bvec > w2)).astype(jnp.int32))

        def retry(_r, _):
            wr = plsc.load_gather(claim_v, [idsl])
            mr = own & (bvec > wr)
            plsc.store_scatter(claim_v, [idsl], bvec, mask=mr)
            return 0
        lax.fori_loop(0, pending, retry, 0)
        return 0
    lax.fori_loop(0, N_GROUPS, claim_grp, 0)

    def compact_grp(g, ptr):
        ids16 = ids_v[pl.ds(g * L, L)]
        own = (ids16 >= lo) & (ids16 < hi)
        bvec = g * L + lane
        idsl = jnp.where(own, ids16 - lo, 0)
        w = plsc.load_gather(claim_v, [idsl])
        win = own & (w == bvec)
        wini = win.astype(jnp.int32)
        pos = ptr + plsc.cumsum(wini) - 1
        plsc.store_scatter(list_v, [pos], bvec, mask=win)
        return ptr + jnp.sum(wini)
    nwin = lax.fori_loop(0, N_GROUPS, compact_grp, jnp.int32(0))

    nchunks = (nwin + CHUNK - 1) // CHUNK

    def pad_grp(g, _):
        pos16 = g * L + lane
        cur = plsc.load_gather(list_v, [jnp.minimum(pos16, BATCH - 1)])
        first = plsc.load_gather(list_v, [jnp.zeros((L,), jnp.int32)])
        plsc.store_scatter(list_v, [pos16], jnp.where(pos16 < nwin, cur, first))
        return 0
    lax.fori_loop(nwin // L, (nchunks * CHUNK) // L, pad_grp, 0)

    def chunk_body(ck, _):
        base = ck * CHUNK
        for g in range(CHUNK // L):
            pos = base + g * L + lane
            bv = plsc.load_gather(list_v, [pos])
            nv = plsc.load_gather(ids_v, [bv])
            tsv = plsc.load_gather(ts_v, [bv])
            prev = plsc.load_gather(lut_v, [nv - lo])
            dt = jnp.maximum(tsv - prev, 0.0)
            av = jnp.exp(-LN2 * dt / HALF_LIFE)
            sl = pl.ds(g * L, L)
            bidx_v[sl] = bv
            nidx_v[sl] = nv
            alpha_v[sl] = av
            tsn_v[sl] = tsv
        cold = pltpu.async_copy(mem_hbm.at[nidx_v], old_v, sem0)
        cns = pltpu.async_copy(ns_hbm.at[bidx_v], nsr_v, sem1)
        cold.wait()
        cns.wait()

        def row_body(r, _):
            a = plsc.load_gather(alpha_v, [jnp.full((L,), r, jnp.int32)])
            for c in range(MEM_DIM // L):
                sl = pl.ds(c * L, L)
                new_v[r, sl] = a * old_v[r, sl] + (1.0 - a) * nsr_v[r, sl]
            return 0
        lax.fori_loop(0, CHUNK, row_body, 0)

        s1 = pltpu.async_copy(new_v, memout_hbm.at[nidx_v], sem0)
        s2 = pltpu.async_copy(tsn_v, tsout_hbm.at[nidx_v], sem1)
        s1.wait()
        s2.wait()
        return 0
    lax.fori_loop(0, nchunks, chunk_body, 0)


@jax.jit
def kernel(node_ids, new_states, ts, memory, last_update_ts):
    mesh = plsc.VectorSubcoreMesh(core_axis_name="c", subcore_axis_name="s",
                                  num_cores=NC, num_subcores=NS)
    f = pl.kernel(
        _sc_body,
        out_type=(jax.ShapeDtypeStruct((N_NODES, MEM_DIM), jnp.float32),
                  jax.ShapeDtypeStruct((N_NODES,), jnp.float32)),
        mesh=mesh,
        scratch_types=(
            pltpu.VMEM((BATCH,), jnp.int32),
            pltpu.VMEM((BATCH,), jnp.float32),
            pltpu.VMEM((ROWS_W,), jnp.float32),
            pltpu.VMEM((((ROWS_W + L - 1) // L) * L,), jnp.int32),
            pltpu.VMEM((BATCH + CHUNK,), jnp.int32),
            pltpu.VMEM((CHUNK,), jnp.int32),
            pltpu.VMEM((CHUNK,), jnp.int32),
            pltpu.VMEM((CHUNK,), jnp.float32),
            pltpu.VMEM((CHUNK,), jnp.float32),
            pltpu.VMEM((CHUNK, MEM_DIM), jnp.float32),
            pltpu.VMEM((CHUNK, MEM_DIM), jnp.float32),
            pltpu.VMEM((CHUNK, MEM_DIM), jnp.float32),
            pltpu.VMEM((CP_ROWS, MEM_DIM), jnp.float32),
            pltpu.SemaphoreType.DMA,
            pltpu.SemaphoreType.DMA,
        ),
        compiler_params=pltpu.CompilerParams(needs_layout_passes=False,
                                             use_tc_tiling_on_sc=False),
    )
    return f(node_ids, new_states, ts, memory, last_update_ts)
